# baseline (speedup 1.0000x reference)
import functools

import numpy as np
import jax
import jax.numpy as jnp
from jax import lax
from jax.experimental import pallas as pl
from jax.experimental.pallas import tpu as pltpu
from jax.experimental.pallas import tpu_sc as plsc

D = 128
H = 8
DK = 16
DFF = 256
N = 10000
E = 320000

NC = 2
NS = 16
NW = NC * NS
RW = 144
KW = 272
EPT = E // NW
CB = 40
NCHUNK = EPT // CB
RPT = 624
RTAIL = N - NS * RPT
TN = 400
GRID = N // TN

_f32 = jnp.float32


def _pre_body(xw, xt, xg,
              cq1, bq1, cv1, bv1,
              cq2, bq2, cv2, bv2,
              qp1, kv1, qp2, kv2):
    xw_ = xw[...]
    xt_ = xt[...]
    xg_ = xg[...]
    qp1[...] = jnp.dot(xt_, cq1[...], preferred_element_type=_f32) + bq1[...]
    kv1[...] = jnp.dot(xw_, cv1[...], preferred_element_type=_f32) + bv1[...]
    qp2[...] = jnp.dot(xg_, cq2[...], preferred_element_type=_f32) + bq2[...]
    kv2[...] = jnp.dot(xt_, cv2[...], preferred_element_type=_f32) + bv2[...]


def _pre_call(xw, xt, xg, t1, t2):
    row = pl.BlockSpec((TN, D), lambda i: (i, 0))
    roww = pl.BlockSpec((TN, RW), lambda i: (i, 0))
    rowkv = pl.BlockSpec((TN, KW), lambda i: (i, 0))
    qfull = pl.BlockSpec((D, RW), lambda i: (0, 0))
    qbfull = pl.BlockSpec((1, RW), lambda i: (0, 0))
    vfull = pl.BlockSpec((D, KW), lambda i: (0, 0))
    vbfull = pl.BlockSpec((1, KW), lambda i: (0, 0))
    wspecs = [qfull, qbfull, vfull, vbfull]
    return pl.pallas_call(
        _pre_body,
        grid=(GRID,),
        in_specs=[row, row, row] + wspecs + wspecs,
        out_specs=[roww, rowkv, roww, rowkv],
        out_shape=[
            jax.ShapeDtypeStruct((N, RW), _f32),
            jax.ShapeDtypeStruct((N, KW), _f32),
            jax.ShapeDtypeStruct((N, RW), _f32),
            jax.ShapeDtypeStruct((N, KW), _f32),
        ],
    )(xw, xt, xg, *t1, *t2)


@functools.lru_cache(maxsize=None)
def _sc_edges_build():
    mesh = plsc.VectorSubcoreMesh(core_axis_name="c", subcore_axis_name="s",
                                  num_cores=NC, num_subcores=NS)
    return functools.partial(
        pl.kernel,
        out_type=jax.ShapeDtypeStruct((2 * N, RW), _f32),
        mesh=mesh,
        compiler_params=pltpu.CompilerParams(use_tc_tiling_on_sc=False,
                                             needs_layout_passes=False),
        scratch_types=[
            pltpu.VMEM((2, CB), jnp.int32),
            pltpu.VMEM((2, CB), jnp.int32),
            pltpu.VMEM((CB,), jnp.int32),
            pltpu.VMEM((2, CB, RW), _f32),
            pltpu.VMEM((2, CB, KW), _f32),
            pltpu.VMEM((CB, RW), _f32),
            pltpu.VMEM_SHARED((N, RW), _f32),
            pltpu.SemaphoreType.DMA,
            pltpu.SemaphoreType.DMA,
            pltpu.SemaphoreType.DMA,
        ],
    )(_sc_edges_body)


def _sc_edges_body(tgt_hbm, src_hbm, qp_hbm, kv_hbm, zero_hbm, out_hbm,
                   idxt_v, idxs_v, scx_v, q_v, kv_v, wv_v, acc_sh,
                   s_ix, s_g, s_sc):
    cid = lax.axis_index("c")
    sid = lax.axis_index("s")
    wid = cid * NS + sid

    pltpu.sync_copy(zero_hbm.at[pl.ds(sid * RPT, RPT)],
                    acc_sh.at[pl.ds(sid * RPT, RPT)])

    @pl.when(sid == 0)
    def _zero_tail():
        pltpu.sync_copy(zero_hbm.at[pl.ds(NS * RPT, RTAIL)],
                        acc_sh.at[pl.ds(NS * RPT, RTAIL)])
    plsc.subcore_barrier()
    lanes = lax.iota(jnp.int32, 16)
    base0 = wid * EPT

    def fetch_idx(i, b):
        pltpu.async_copy(tgt_hbm.at[pl.ds(base0 + i * CB, CB)],
                         idxt_v.at[b], s_ix)
        pltpu.async_copy(src_hbm.at[pl.ds(base0 + i * CB, CB)],
                         idxs_v.at[b], s_ix)

    def drain_idx(b):
        pltpu.make_async_copy(tgt_hbm.at[pl.ds(0, CB)], idxt_v.at[b],
                              s_ix).wait()
        pltpu.make_async_copy(src_hbm.at[pl.ds(0, CB)], idxs_v.at[b],
                              s_ix).wait()

    def issue_gather(b):
        pltpu.async_copy(qp_hbm.at[idxt_v.at[b]], q_v.at[b], s_g)
        pltpu.async_copy(kv_hbm.at[idxs_v.at[b]], kv_v.at[b], s_g)

    def drain_gather(b):
        pltpu.make_async_copy(qp_hbm.at[idxt_v.at[b]], q_v.at[b], s_g).wait()
        pltpu.make_async_copy(kv_hbm.at[idxs_v.at[b]], kv_v.at[b], s_g).wait()

    def drain_scatter():
        pltpu.make_async_copy(wv_v, acc_sh.at[scx_v], s_sc).wait()

    def compute(b):
        @plsc.parallel_loop(0, CB, unroll=4)
        def edge(e):
            svec = jnp.zeros((16,), _f32)
            for h in range(H):
                s = jnp.sum(q_v[b, e, pl.ds(h * DK, DK)]
                            * kv_v[b, e, pl.ds(D + h * DK, DK)])
                svec = jnp.where(lanes == h, s, svec)
            ex = jnp.exp(svec + q_v[b, e, pl.ds(D, DK)]
                         + kv_v[b, e, pl.ds(2 * D, DK)])
            wv_v[e, pl.ds(D, DK)] = ex
            for h in range(H):
                eh = ex.at[jnp.full((16,), h, jnp.int32)].get(
                    mode="promise_in_bounds")
                wv_v[e, pl.ds(h * DK, DK)] = (
                    eh * kv_v[b, e, pl.ds(h * DK, DK)])

    def save_scatter_idx(b):
        scx_v[pl.ds(0, 16)] = idxt_v[b, pl.ds(0, 16)]
        scx_v[pl.ds(16, 16)] = idxt_v[b, pl.ds(16, 16)]
        scx_v[pl.ds(CB - 16, 16)] = idxt_v[b, pl.ds(CB - 16, 16)]

    def half(i, b, t, first):
        drain_gather(b)

        @pl.when(i + 1 < NCHUNK)
        def _issue_next():
            drain_idx(1 - b)
            issue_gather(1 - b)
        if first:
            @pl.when(t > 0)
            def _drain_sc():
                drain_scatter()
        else:
            drain_scatter()
        save_scatter_idx(b)

        @pl.when(t < NCHUNK // 2 - 1)
        def _prefetch():
            fetch_idx(i + 2, b)
        compute(b)
        pltpu.async_copy(wv_v, acc_sh.at[scx_v], s_sc, add=True)

    pltpu.sync_copy(tgt_hbm.at[pl.ds(base0, CB)], idxt_v.at[0])
    pltpu.sync_copy(src_hbm.at[pl.ds(base0, CB)], idxs_v.at[0])
    issue_gather(0)
    fetch_idx(1, 1)

    def pair(t, carry):
        i0 = 2 * t
        half(i0, 0, t, True)
        half(i0 + 1, 1, t, False)
        return carry

    lax.fori_loop(0, NCHUNK // 2, pair, 0)
    drain_scatter()
    plsc.subcore_barrier()
    pltpu.sync_copy(acc_sh.at[pl.ds(sid * RPT, RPT)],
                    out_hbm.at[pl.ds(cid * N + sid * RPT, RPT)])

    @pl.when(sid == 0)
    def _flush_tail():
        pltpu.sync_copy(acc_sh.at[pl.ds(NS * RPT, RTAIL)],
                        out_hbm.at[pl.ds(cid * N + NS * RPT, RTAIL)])


_REXP = np.kron(np.eye(H, dtype=np.float32), np.ones((1, DK), np.float32))


def _ln(x, g, b):
    m = jnp.mean(x, axis=-1, keepdims=True)
    v = jnp.mean((x - m) ** 2, axis=-1, keepdims=True)
    return (x - m) * lax.rsqrt(v + 1e-5) * g + b


def _post_body(a1a, a1b, a2a, a2b, xw, xt, xg, rexp,
               owt, obt, owg, obg,
               ln1g, ln1b, f1w, f1b, f2w, f2b, ln2g, ln2b,
               yw, yt, yg):
    def norm(ahi, alo):
        acc = ahi[...] + alo[...]
        s = acc[:, D:D + H]
        r = jnp.where(s > 0.0, 1.0 / s, 0.0)
        return acc[:, :D] * jnp.dot(r, rexp[...], preferred_element_type=_f32)

    msg1 = norm(a1a, a1b)
    msg2 = norm(a2a, a2b)
    xw_ = xw[...]
    xt_ = xt[...]
    xg_ = xg[...]
    h_w = _ln(xw_, ln1g[0], ln1b[0])
    h_t = _ln(xt_ + jnp.dot(msg1, owt[...], preferred_element_type=_f32)
              + obt[...], ln1g[1], ln1b[1])
    h_g = _ln(xg_ + jnp.dot(msg2, owg[...], preferred_element_type=_f32)
              + obg[...], ln1g[2], ln1b[2])
    for i, (hh, yref) in enumerate(((h_w, yw), (h_t, yt), (h_g, yg))):
        f = jnp.maximum(
            jnp.dot(hh, f1w[i], preferred_element_type=_f32) + f1b[i], 0.0)
        f = jnp.dot(f, f2w[i], preferred_element_type=_f32) + f2b[i]
        yref[...] = _ln(hh + f, ln2g[i], ln2b[i])


def _post_call(acc1, acc2, xw, xt, xg, pp):
    rowa = pl.BlockSpec((TN, RW), lambda i: (i, 0))
    rowb = pl.BlockSpec((TN, RW), lambda i: (i + GRID, 0))
    row = pl.BlockSpec((TN, D), lambda i: (i, 0))

    def full(*shape):
        return pl.BlockSpec(shape, lambda i, _n=len(shape): (0,) * _n)

    in_specs = [rowa, rowb, rowa, rowb, row, row, row,
                full(H, D),
                full(D, D), full(1, D), full(D, D), full(1, D),
                full(3, D), full(3, D),
                full(3, D, DFF), full(3, 1, DFF),
                full(3, DFF, D), full(3, 1, D),
                full(3, D), full(3, D)]
    return pl.pallas_call(
        _post_body,
        grid=(GRID,),
        in_specs=in_specs,
        out_specs=[row, row, row],
        out_shape=[jax.ShapeDtypeStruct((N, D), _f32)] * 3,
    )(acc1, acc1, acc2, acc2, xw, xt, xg, jnp.asarray(_REXP), *pp)


def _tables(tp, sp, ep):
    w = ep["w"]
    zc = jnp.zeros((D, H), _f32)
    cq = jnp.concatenate([tp["q"]["w"].T * 0.25, w[:, D:].T, zc], axis=1)
    bq = jnp.concatenate([tp["q"]["b"] * 0.25, ep["b"],
                          jnp.zeros((H,), _f32)])[None]
    cv = jnp.concatenate([sp["v"]["w"].T, sp["k"]["w"].T, w[:, :D].T, zc],
                         axis=1)
    bv = jnp.concatenate([sp["v"]["b"], sp["k"]["b"],
                          jnp.zeros((2 * H,), _f32)])[None]
    return cq, bq, cv, bv


def kernel(x_wave, x_transition, x_target, edge_index_wt, edge_index_tt,
           params):
    t1 = _tables(params["transition"], params["wave"], params["edge_wt"])
    t2 = _tables(params["target"], params["transition"], params["edge_tt"])
    qp1, kv1, qp2, kv2 = _pre_call(x_wave, x_transition, x_target, t1, t2)
    zero = jnp.zeros((N, RW), _f32)
    sc_edges = _sc_edges_build()
    acc1 = sc_edges(edge_index_wt[0], edge_index_wt[1], qp1, kv1, zero)
    acc2 = sc_edges(edge_index_tt[0], edge_index_tt[1], qp2, kv2, zero)

    order = ("wave", "transition", "target")
    pp = [
        params["transition"]["out"]["w"].T,
        params["transition"]["out"]["b"][None],
        params["target"]["out"]["w"].T,
        params["target"]["out"]["b"][None],
        jnp.stack([params[t]["ln1_g"] for t in order]),
        jnp.stack([params[t]["ln1_b"] for t in order]),
        jnp.stack([params[t]["ffn1"]["w"].T for t in order]),
        jnp.stack([params[t]["ffn1"]["b"][None] for t in order]),
        jnp.stack([params[t]["ffn2"]["w"].T for t in order]),
        jnp.stack([params[t]["ffn2"]["b"][None] for t in order]),
        jnp.stack([params[t]["ln2_g"] for t in order]),
        jnp.stack([params[t]["ln2_b"] for t in order]),
    ]
    yw, yt, yg = _post_call(acc1, acc2, x_wave, x_transition, x_target, pp)
    return (yw, yt, yg)


# speedup vs baseline: 1.0011x; 1.0011x over previous
import functools

import numpy as np
import jax
import jax.numpy as jnp
from jax import lax
from jax.experimental import pallas as pl
from jax.experimental.pallas import tpu as pltpu
from jax.experimental.pallas import tpu_sc as plsc

D = 128
H = 8
DK = 16
DFF = 256
N = 10000
E = 320000

NC = 2
NS = 16
NW = NC * NS
RW = 144
KW = 272
EPT = E // NW
CB = 40
NCHUNK = EPT // CB
RPT = 624
RTAIL = N - NS * RPT
TN = 400
GRID = N // TN

_f32 = jnp.float32


def _pre_body(xw, xt, xg,
              cq1, bq1, cv1, bv1,
              cq2, bq2, cv2, bv2,
              qp1, kv1, qp2, kv2):
    xw_ = xw[...]
    xt_ = xt[...]
    xg_ = xg[...]
    qp1[...] = jnp.dot(xt_, cq1[...], preferred_element_type=_f32) + bq1[...]
    kv1[...] = jnp.dot(xw_, cv1[...], preferred_element_type=_f32) + bv1[...]
    qp2[...] = jnp.dot(xg_, cq2[...], preferred_element_type=_f32) + bq2[...]
    kv2[...] = jnp.dot(xt_, cv2[...], preferred_element_type=_f32) + bv2[...]


def _pre_call(xw, xt, xg, t1, t2):
    row = pl.BlockSpec((TN, D), lambda i: (i, 0))
    roww = pl.BlockSpec((TN, RW), lambda i: (i, 0))
    rowkv = pl.BlockSpec((TN, KW), lambda i: (i, 0))
    qfull = pl.BlockSpec((D, RW), lambda i: (0, 0))
    qbfull = pl.BlockSpec((1, RW), lambda i: (0, 0))
    vfull = pl.BlockSpec((D, KW), lambda i: (0, 0))
    vbfull = pl.BlockSpec((1, KW), lambda i: (0, 0))
    wspecs = [qfull, qbfull, vfull, vbfull]
    return pl.pallas_call(
        _pre_body,
        grid=(GRID,),
        in_specs=[row, row, row] + wspecs + wspecs,
        out_specs=[roww, rowkv, roww, rowkv],
        out_shape=[
            jax.ShapeDtypeStruct((N, RW), _f32),
            jax.ShapeDtypeStruct((N, KW), _f32),
            jax.ShapeDtypeStruct((N, RW), _f32),
            jax.ShapeDtypeStruct((N, KW), _f32),
        ],
    )(xw, xt, xg, *t1, *t2)


@functools.lru_cache(maxsize=None)
def _sc_edges_build():
    mesh = plsc.VectorSubcoreMesh(core_axis_name="c", subcore_axis_name="s",
                                  num_cores=NC, num_subcores=NS)
    return functools.partial(
        pl.kernel,
        out_type=jax.ShapeDtypeStruct((2 * N, RW), _f32),
        mesh=mesh,
        compiler_params=pltpu.CompilerParams(use_tc_tiling_on_sc=False,
                                             needs_layout_passes=False),
        scratch_types=[
            pltpu.VMEM((2, CB), jnp.int32),
            pltpu.VMEM((2, CB), jnp.int32),
            pltpu.VMEM((CB,), jnp.int32),
            pltpu.VMEM((2, CB, RW), _f32),
            pltpu.VMEM((2, CB, KW), _f32),
            pltpu.VMEM((CB, RW), _f32),
            pltpu.VMEM_SHARED((N, RW), _f32),
            pltpu.SemaphoreType.DMA,
            pltpu.SemaphoreType.DMA,
            pltpu.SemaphoreType.DMA,
        ],
    )(_sc_edges_body)


def _sc_edges_body(tgt_hbm, src_hbm, qp_hbm, kv_hbm, zero_hbm, out_hbm,
                   idxt_v, idxs_v, scx_v, q_v, kv_v, wv_v, acc_sh,
                   s_ix, s_g, s_sc):
    cid = lax.axis_index("c")
    sid = lax.axis_index("s")
    wid = cid * NS + sid

    pltpu.sync_copy(zero_hbm.at[pl.ds(sid * RPT, RPT)],
                    acc_sh.at[pl.ds(sid * RPT, RPT)])

    @pl.when(sid == 0)
    def _zero_tail():
        pltpu.sync_copy(zero_hbm.at[pl.ds(NS * RPT, RTAIL)],
                        acc_sh.at[pl.ds(NS * RPT, RTAIL)])
    plsc.subcore_barrier()
    lanes = lax.iota(jnp.int32, 16)
    base0 = wid * EPT

    def fetch_idx(i, b):
        pltpu.async_copy(tgt_hbm.at[pl.ds(base0 + i * CB, CB)],
                         idxt_v.at[b], s_ix)
        pltpu.async_copy(src_hbm.at[pl.ds(base0 + i * CB, CB)],
                         idxs_v.at[b], s_ix)

    def drain_idx(b):
        pltpu.make_async_copy(tgt_hbm.at[pl.ds(0, CB)], idxt_v.at[b],
                              s_ix).wait()
        pltpu.make_async_copy(src_hbm.at[pl.ds(0, CB)], idxs_v.at[b],
                              s_ix).wait()

    def issue_gather(b):
        pltpu.async_copy(qp_hbm.at[idxt_v.at[b]], q_v.at[b], s_g)
        pltpu.async_copy(kv_hbm.at[idxs_v.at[b]], kv_v.at[b], s_g)

    def drain_gather(b):
        pltpu.make_async_copy(qp_hbm.at[idxt_v.at[b]], q_v.at[b], s_g).wait()
        pltpu.make_async_copy(kv_hbm.at[idxs_v.at[b]], kv_v.at[b], s_g).wait()

    def drain_scatter():
        pltpu.make_async_copy(wv_v, acc_sh.at[scx_v], s_sc).wait()

    def compute(b):
        @plsc.parallel_loop(0, CB, unroll=2)
        def edge(e):
            svec = jnp.zeros((16,), _f32)
            for h in range(H):
                s = jnp.sum(q_v[b, e, pl.ds(h * DK, DK)]
                            * kv_v[b, e, pl.ds(D + h * DK, DK)])
                svec = jnp.where(lanes == h, s, svec)
            ex = jnp.exp(svec + q_v[b, e, pl.ds(D, DK)]
                         + kv_v[b, e, pl.ds(2 * D, DK)])
            wv_v[e, pl.ds(D, DK)] = ex
            for h in range(H):
                eh = ex.at[jnp.full((16,), h, jnp.int32)].get(
                    mode="promise_in_bounds")
                wv_v[e, pl.ds(h * DK, DK)] = (
                    eh * kv_v[b, e, pl.ds(h * DK, DK)])

    def save_scatter_idx(b):
        scx_v[pl.ds(0, 16)] = idxt_v[b, pl.ds(0, 16)]
        scx_v[pl.ds(16, 16)] = idxt_v[b, pl.ds(16, 16)]
        scx_v[pl.ds(CB - 16, 16)] = idxt_v[b, pl.ds(CB - 16, 16)]

    def half(i, b, t, first):
        drain_gather(b)

        @pl.when(i + 1 < NCHUNK)
        def _issue_next():
            drain_idx(1 - b)
            issue_gather(1 - b)
        if first:
            @pl.when(t > 0)
            def _drain_sc():
                drain_scatter()
        else:
            drain_scatter()
        save_scatter_idx(b)

        @pl.when(t < NCHUNK // 2 - 1)
        def _prefetch():
            fetch_idx(i + 2, b)
        compute(b)
        pltpu.async_copy(wv_v, acc_sh.at[scx_v], s_sc, add=True)

    pltpu.sync_copy(tgt_hbm.at[pl.ds(base0, CB)], idxt_v.at[0])
    pltpu.sync_copy(src_hbm.at[pl.ds(base0, CB)], idxs_v.at[0])
    issue_gather(0)
    fetch_idx(1, 1)

    def pair(t, carry):
        i0 = 2 * t
        half(i0, 0, t, True)
        half(i0 + 1, 1, t, False)
        return carry

    lax.fori_loop(0, NCHUNK // 2, pair, 0)
    drain_scatter()
    plsc.subcore_barrier()
    pltpu.sync_copy(acc_sh.at[pl.ds(sid * RPT, RPT)],
                    out_hbm.at[pl.ds(cid * N + sid * RPT, RPT)])

    @pl.when(sid == 0)
    def _flush_tail():
        pltpu.sync_copy(acc_sh.at[pl.ds(NS * RPT, RTAIL)],
                        out_hbm.at[pl.ds(cid * N + NS * RPT, RTAIL)])


_REXP = np.kron(np.eye(H, dtype=np.float32), np.ones((1, DK), np.float32))


def _ln(x, g, b):
    m = jnp.mean(x, axis=-1, keepdims=True)
    v = jnp.mean((x - m) ** 2, axis=-1, keepdims=True)
    return (x - m) * lax.rsqrt(v + 1e-5) * g + b


def _post_body(a1a, a1b, a2a, a2b, xw, xt, xg, rexp,
               owt, obt, owg, obg,
               ln1g, ln1b, f1w, f1b, f2w, f2b, ln2g, ln2b,
               yw, yt, yg):
    def norm(ahi, alo):
        acc = ahi[...] + alo[...]
        s = acc[:, D:D + H]
        r = jnp.where(s > 0.0, 1.0 / s, 0.0)
        return acc[:, :D] * jnp.dot(r, rexp[...], preferred_element_type=_f32)

    msg1 = norm(a1a, a1b)
    msg2 = norm(a2a, a2b)
    xw_ = xw[...]
    xt_ = xt[...]
    xg_ = xg[...]
    h_w = _ln(xw_, ln1g[0], ln1b[0])
    h_t = _ln(xt_ + jnp.dot(msg1, owt[...], preferred_element_type=_f32)
              + obt[...], ln1g[1], ln1b[1])
    h_g = _ln(xg_ + jnp.dot(msg2, owg[...], preferred_element_type=_f32)
              + obg[...], ln1g[2], ln1b[2])
    for i, (hh, yref) in enumerate(((h_w, yw), (h_t, yt), (h_g, yg))):
        f = jnp.maximum(
            jnp.dot(hh, f1w[i], preferred_element_type=_f32) + f1b[i], 0.0)
        f = jnp.dot(f, f2w[i], preferred_element_type=_f32) + f2b[i]
        yref[...] = _ln(hh + f, ln2g[i], ln2b[i])


def _post_call(acc1, acc2, xw, xt, xg, pp):
    rowa = pl.BlockSpec((TN, RW), lambda i: (i, 0))
    rowb = pl.BlockSpec((TN, RW), lambda i: (i + GRID, 0))
    row = pl.BlockSpec((TN, D), lambda i: (i, 0))

    def full(*shape):
        return pl.BlockSpec(shape, lambda i, _n=len(shape): (0,) * _n)

    in_specs = [rowa, rowb, rowa, rowb, row, row, row,
                full(H, D),
                full(D, D), full(1, D), full(D, D), full(1, D),
                full(3, D), full(3, D),
                full(3, D, DFF), full(3, 1, DFF),
                full(3, DFF, D), full(3, 1, D),
                full(3, D), full(3, D)]
    return pl.pallas_call(
        _post_body,
        grid=(GRID,),
        in_specs=in_specs,
        out_specs=[row, row, row],
        out_shape=[jax.ShapeDtypeStruct((N, D), _f32)] * 3,
    )(acc1, acc1, acc2, acc2, xw, xt, xg, jnp.asarray(_REXP), *pp)


def _tables(tp, sp, ep):
    w = ep["w"]
    zc = jnp.zeros((D, H), _f32)
    cq = jnp.concatenate([tp["q"]["w"].T * 0.25, w[:, D:].T, zc], axis=1)
    bq = jnp.concatenate([tp["q"]["b"] * 0.25, ep["b"],
                          jnp.zeros((H,), _f32)])[None]
    cv = jnp.concatenate([sp["v"]["w"].T, sp["k"]["w"].T, w[:, :D].T, zc],
                         axis=1)
    bv = jnp.concatenate([sp["v"]["b"], sp["k"]["b"],
                          jnp.zeros((2 * H,), _f32)])[None]
    return cq, bq, cv, bv


def kernel(x_wave, x_transition, x_target, edge_index_wt, edge_index_tt,
           params):
    t1 = _tables(params["transition"], params["wave"], params["edge_wt"])
    t2 = _tables(params["target"], params["transition"], params["edge_tt"])
    qp1, kv1, qp2, kv2 = _pre_call(x_wave, x_transition, x_target, t1, t2)
    zero = jnp.zeros((N, RW), _f32)
    sc_edges = _sc_edges_build()
    acc1 = sc_edges(edge_index_wt[0], edge_index_wt[1], qp1, kv1, zero)
    acc2 = sc_edges(edge_index_tt[0], edge_index_tt[1], qp2, kv2, zero)

    order = ("wave", "transition", "target")
    pp = [
        params["transition"]["out"]["w"].T,
        params["transition"]["out"]["b"][None],
        params["target"]["out"]["w"].T,
        params["target"]["out"]["b"][None],
        jnp.stack([params[t]["ln1_g"] for t in order]),
        jnp.stack([params[t]["ln1_b"] for t in order]),
        jnp.stack([params[t]["ffn1"]["w"].T for t in order]),
        jnp.stack([params[t]["ffn1"]["b"][None] for t in order]),
        jnp.stack([params[t]["ffn2"]["w"].T for t in order]),
        jnp.stack([params[t]["ffn2"]["b"][None] for t in order]),
        jnp.stack([params[t]["ln2_g"] for t in order]),
        jnp.stack([params[t]["ln2_b"] for t in order]),
    ]
    yw, yt, yg = _post_call(acc1, acc2, x_wave, x_transition, x_target, pp)
    return (yw, yt, yg)


# speedup vs baseline: 1.0125x; 1.0114x over previous
import functools

import numpy as np
import jax
import jax.numpy as jnp
from jax import lax
from jax.experimental import pallas as pl
from jax.experimental.pallas import tpu as pltpu
from jax.experimental.pallas import tpu_sc as plsc

D = 128
H = 8
DK = 16
DFF = 256
N = 10000
E = 320000

NC = 2
NS = 16
NW = NC * NS
RW = 144
KW = 272
EPT = E // NW
CB = 40
NCHUNK = EPT // CB
RPT = 624
RTAIL = N - NS * RPT
TN = 400
GRID = N // TN

_f32 = jnp.float32


def _pre_body(xs, xt, cq, bq, cv, bv, qp, kv):
    qp[...] = jnp.dot(xt[...], cq[...], preferred_element_type=_f32) + bq[...]
    kv[...] = jnp.dot(xs[...], cv[...], preferred_element_type=_f32) + bv[...]


def _pre_call(xs, xt, t):
    row = pl.BlockSpec((TN, D), lambda i: (i, 0))
    roww = pl.BlockSpec((TN, RW), lambda i: (i, 0))
    rowkv = pl.BlockSpec((TN, KW), lambda i: (i, 0))
    qfull = pl.BlockSpec((D, RW), lambda i: (0, 0))
    qbfull = pl.BlockSpec((1, RW), lambda i: (0, 0))
    vfull = pl.BlockSpec((D, KW), lambda i: (0, 0))
    vbfull = pl.BlockSpec((1, KW), lambda i: (0, 0))
    return pl.pallas_call(
        _pre_body,
        grid=(GRID,),
        in_specs=[row, row, qfull, qbfull, vfull, vbfull],
        out_specs=[roww, rowkv],
        out_shape=[
            jax.ShapeDtypeStruct((N, RW), _f32),
            jax.ShapeDtypeStruct((N, KW), _f32),
        ],
    )(xs, xt, *t)


@functools.lru_cache(maxsize=None)
def _sc_edges_build():
    mesh = plsc.VectorSubcoreMesh(core_axis_name="c", subcore_axis_name="s",
                                  num_cores=NC, num_subcores=NS)
    return functools.partial(
        pl.kernel,
        out_type=jax.ShapeDtypeStruct((2 * N, RW), _f32),
        mesh=mesh,
        compiler_params=pltpu.CompilerParams(use_tc_tiling_on_sc=False,
                                             needs_layout_passes=False),
        scratch_types=[
            pltpu.VMEM((2, CB), jnp.int32),
            pltpu.VMEM((2, CB), jnp.int32),
            pltpu.VMEM((CB,), jnp.int32),
            pltpu.VMEM((2, CB, RW), _f32),
            pltpu.VMEM((2, CB, KW), _f32),
            pltpu.VMEM((CB, RW), _f32),
            pltpu.VMEM_SHARED((N, RW), _f32),
            pltpu.SemaphoreType.DMA,
            pltpu.SemaphoreType.DMA,
            pltpu.SemaphoreType.DMA,
        ],
    )(_sc_edges_body)


def _sc_edges_body(tgt_hbm, src_hbm, qp_hbm, kv_hbm, zero_hbm, out_hbm,
                   idxt_v, idxs_v, scx_v, q_v, kv_v, wv_v, acc_sh,
                   s_ix, s_g, s_sc):
    cid = lax.axis_index("c")
    sid = lax.axis_index("s")
    wid = cid * NS + sid

    pltpu.sync_copy(zero_hbm.at[pl.ds(sid * RPT, RPT)],
                    acc_sh.at[pl.ds(sid * RPT, RPT)])

    @pl.when(sid == 0)
    def _zero_tail():
        pltpu.sync_copy(zero_hbm.at[pl.ds(NS * RPT, RTAIL)],
                        acc_sh.at[pl.ds(NS * RPT, RTAIL)])
    plsc.subcore_barrier()
    lanes = lax.iota(jnp.int32, 16)
    base0 = wid * EPT

    def fetch_idx(i, b):
        pltpu.async_copy(tgt_hbm.at[pl.ds(base0 + i * CB, CB)],
                         idxt_v.at[b], s_ix)
        pltpu.async_copy(src_hbm.at[pl.ds(base0 + i * CB, CB)],
                         idxs_v.at[b], s_ix)

    def drain_idx(b):
        pltpu.make_async_copy(tgt_hbm.at[pl.ds(0, CB)], idxt_v.at[b],
                              s_ix).wait()
        pltpu.make_async_copy(src_hbm.at[pl.ds(0, CB)], idxs_v.at[b],
                              s_ix).wait()

    def issue_gather(b):
        pltpu.async_copy(qp_hbm.at[idxt_v.at[b]], q_v.at[b], s_g)
        pltpu.async_copy(kv_hbm.at[idxs_v.at[b]], kv_v.at[b], s_g)

    def drain_gather(b):
        pltpu.make_async_copy(qp_hbm.at[idxt_v.at[b]], q_v.at[b], s_g).wait()
        pltpu.make_async_copy(kv_hbm.at[idxs_v.at[b]], kv_v.at[b], s_g).wait()

    def drain_scatter():
        pltpu.make_async_copy(wv_v, acc_sh.at[scx_v], s_sc).wait()

    def compute(b):
        @plsc.parallel_loop(0, CB, unroll=2)
        def edge(e):
            svec = jnp.zeros((16,), _f32)
            for h in range(H):
                s = jnp.sum(q_v[b, e, pl.ds(h * DK, DK)]
                            * kv_v[b, e, pl.ds(D + h * DK, DK)])
                svec = jnp.where(lanes == h, s, svec)
            ex = jnp.exp(svec + q_v[b, e, pl.ds(D, DK)]
                         + kv_v[b, e, pl.ds(2 * D, DK)])
            wv_v[e, pl.ds(D, DK)] = ex
            for h in range(H):
                eh = ex.at[jnp.full((16,), h, jnp.int32)].get(
                    mode="promise_in_bounds")
                wv_v[e, pl.ds(h * DK, DK)] = (
                    eh * kv_v[b, e, pl.ds(h * DK, DK)])

    def save_scatter_idx(b):
        scx_v[pl.ds(0, 16)] = idxt_v[b, pl.ds(0, 16)]
        scx_v[pl.ds(16, 16)] = idxt_v[b, pl.ds(16, 16)]
        scx_v[pl.ds(CB - 16, 16)] = idxt_v[b, pl.ds(CB - 16, 16)]

    def half(i, b, t, first):
        drain_gather(b)

        @pl.when(i + 1 < NCHUNK)
        def _issue_next():
            drain_idx(1 - b)
            issue_gather(1 - b)
        if first:
            @pl.when(t > 0)
            def _drain_sc():
                drain_scatter()
        else:
            drain_scatter()
        save_scatter_idx(b)

        @pl.when(t < NCHUNK // 2 - 1)
        def _prefetch():
            fetch_idx(i + 2, b)
        compute(b)
        pltpu.async_copy(wv_v, acc_sh.at[scx_v], s_sc, add=True)

    pltpu.sync_copy(tgt_hbm.at[pl.ds(base0, CB)], idxt_v.at[0])
    pltpu.sync_copy(src_hbm.at[pl.ds(base0, CB)], idxs_v.at[0])
    issue_gather(0)
    fetch_idx(1, 1)

    def pair(t, carry):
        i0 = 2 * t
        half(i0, 0, t, True)
        half(i0 + 1, 1, t, False)
        return carry

    lax.fori_loop(0, NCHUNK // 2, pair, 0)
    drain_scatter()
    plsc.subcore_barrier()
    pltpu.sync_copy(acc_sh.at[pl.ds(sid * RPT, RPT)],
                    out_hbm.at[pl.ds(cid * N + sid * RPT, RPT)])

    @pl.when(sid == 0)
    def _flush_tail():
        pltpu.sync_copy(acc_sh.at[pl.ds(NS * RPT, RTAIL)],
                        out_hbm.at[pl.ds(cid * N + NS * RPT, RTAIL)])


_REXP = np.kron(np.eye(H, dtype=np.float32), np.ones((1, DK), np.float32))


def _ln(x, g, b):
    m = jnp.mean(x, axis=-1, keepdims=True)
    v = jnp.mean((x - m) ** 2, axis=-1, keepdims=True)
    return (x - m) * lax.rsqrt(v + 1e-5) * g + b


def _post_body(a1a, a1b, a2a, a2b, xw, xt, xg, rexp,
               owt, obt, owg, obg,
               ln1g, ln1b, f1w, f1b, f2w, f2b, ln2g, ln2b,
               yw, yt, yg):
    def norm(ahi, alo):
        acc = ahi[...] + alo[...]
        s = acc[:, D:D + H]
        r = jnp.where(s > 0.0, 1.0 / s, 0.0)
        return acc[:, :D] * jnp.dot(r, rexp[...], preferred_element_type=_f32)

    msg1 = norm(a1a, a1b)
    msg2 = norm(a2a, a2b)
    xw_ = xw[...]
    xt_ = xt[...]
    xg_ = xg[...]
    h_w = _ln(xw_, ln1g[0], ln1b[0])
    h_t = _ln(xt_ + jnp.dot(msg1, owt[...], preferred_element_type=_f32)
              + obt[...], ln1g[1], ln1b[1])
    h_g = _ln(xg_ + jnp.dot(msg2, owg[...], preferred_element_type=_f32)
              + obg[...], ln1g[2], ln1b[2])
    for i, (hh, yref) in enumerate(((h_w, yw), (h_t, yt), (h_g, yg))):
        f = jnp.maximum(
            jnp.dot(hh, f1w[i], preferred_element_type=_f32) + f1b[i], 0.0)
        f = jnp.dot(f, f2w[i], preferred_element_type=_f32) + f2b[i]
        yref[...] = _ln(hh + f, ln2g[i], ln2b[i])


def _post_call(acc1, acc2, xw, xt, xg, pp):
    rowa = pl.BlockSpec((TN, RW), lambda i: (i, 0))
    rowb = pl.BlockSpec((TN, RW), lambda i: (i + GRID, 0))
    row = pl.BlockSpec((TN, D), lambda i: (i, 0))

    def full(*shape):
        return pl.BlockSpec(shape, lambda i, _n=len(shape): (0,) * _n)

    in_specs = [rowa, rowb, rowa, rowb, row, row, row,
                full(H, D),
                full(D, D), full(1, D), full(D, D), full(1, D),
                full(3, D), full(3, D),
                full(3, D, DFF), full(3, 1, DFF),
                full(3, DFF, D), full(3, 1, D),
                full(3, D), full(3, D)]
    return pl.pallas_call(
        _post_body,
        grid=(GRID,),
        in_specs=in_specs,
        out_specs=[row, row, row],
        out_shape=[jax.ShapeDtypeStruct((N, D), _f32)] * 3,
    )(acc1, acc1, acc2, acc2, xw, xt, xg, jnp.asarray(_REXP), *pp)


def _tables(tp, sp, ep):
    w = ep["w"]
    zc = jnp.zeros((D, H), _f32)
    cq = jnp.concatenate([tp["q"]["w"].T * 0.25, w[:, D:].T, zc], axis=1)
    bq = jnp.concatenate([tp["q"]["b"] * 0.25, ep["b"],
                          jnp.zeros((H,), _f32)])[None]
    cv = jnp.concatenate([sp["v"]["w"].T, sp["k"]["w"].T, w[:, :D].T, zc],
                         axis=1)
    bv = jnp.concatenate([sp["v"]["b"], sp["k"]["b"],
                          jnp.zeros((2 * H,), _f32)])[None]
    return cq, bq, cv, bv


def kernel(x_wave, x_transition, x_target, edge_index_wt, edge_index_tt,
           params):
    t1 = _tables(params["transition"], params["wave"], params["edge_wt"])
    t2 = _tables(params["target"], params["transition"], params["edge_tt"])
    zero = jnp.zeros((N, RW), _f32)
    sc_edges = _sc_edges_build()
    qp1, kv1 = _pre_call(x_wave, x_transition, t1)
    acc1 = sc_edges(edge_index_wt[0], edge_index_wt[1], qp1, kv1, zero)
    qp2, kv2 = _pre_call(x_transition, x_target, t2)
    acc2 = sc_edges(edge_index_tt[0], edge_index_tt[1], qp2, kv2, zero)

    order = ("wave", "transition", "target")
    pp = [
        params["transition"]["out"]["w"].T,
        params["transition"]["out"]["b"][None],
        params["target"]["out"]["w"].T,
        params["target"]["out"]["b"][None],
        jnp.stack([params[t]["ln1_g"] for t in order]),
        jnp.stack([params[t]["ln1_b"] for t in order]),
        jnp.stack([params[t]["ffn1"]["w"].T for t in order]),
        jnp.stack([params[t]["ffn1"]["b"][None] for t in order]),
        jnp.stack([params[t]["ffn2"]["w"].T for t in order]),
        jnp.stack([params[t]["ffn2"]["b"][None] for t in order]),
        jnp.stack([params[t]["ln2_g"] for t in order]),
        jnp.stack([params[t]["ln2_b"] for t in order]),
    ]
    yw, yt, yg = _post_call(acc1, acc2, x_wave, x_transition, x_target, pp)
    return (yw, yt, yg)
